# baseline (speedup 1.0000x reference)
import functools

import jax
import jax.numpy as jnp
from jax import lax
from jax.experimental import pallas as pl
from jax.experimental.pallas import tpu as pltpu
from jax.experimental.pallas import tpu_sc as plsc

HIDDEN = 128
EPS = 1e-5
LANES = 16
NREG = HIDDEN // LANES
CHUNK = 128


def _lane_sum(v):
    idx = lax.iota(jnp.int32, LANES)
    dnums = lax.GatherDimensionNumbers(
        offset_dims=(), collapsed_slice_dims=(0,), start_index_map=(0,))
    for sh in (8, 4, 2, 1):
        perm = (idx ^ sh).reshape(LANES, 1)
        v = v + lax.gather(v, perm, dnums, (1,),
                           mode=lax.GatherScatterMode.PROMISE_IN_BOUNDS)
    return v


def _rsqrt(v):
    i = lax.bitcast_convert_type(v, jnp.int32)
    i = jnp.int32(0x5F3759DF) - lax.shift_right_logical(i, 1)
    y = lax.bitcast_convert_type(i, jnp.float32)
    half = v * 0.5
    for _ in range(2):
        y = y * (1.5 - half * y * y)
    return y


def _sc_body(table, ids, maskf, gamma, beta, out,
             idx_v, m_v, in0, in1, out0, out1, g_v, b_v,
             gs0, gs1, os0, os1):
    info = plsc.get_sparse_core_info()
    nc = info.num_cores
    wid = lax.axis_index("s") * nc + lax.axis_index("c")
    n_tok = ids.shape[0] * ids.shape[1]
    n_per_w = n_tok // (nc * info.num_subcores)
    n_chunks = n_per_w // CHUNK
    base = pl.multiple_of(wid * n_per_w, CHUNK)
    base_row = pl.multiple_of(wid * (n_per_w // CHUNK), 8)

    pltpu.sync_copy(gamma, g_v)
    pltpu.sync_copy(beta, b_v)
    pltpu.sync_copy(ids.at[pl.ds(base_row, n_chunks)], idx_v)
    pltpu.sync_copy(maskf.at[pl.ds(base, n_per_w)], m_v)
    g_regs = [g_v[pl.ds(16 * j, 16)] for j in range(NREG)]
    b_regs = [b_v[pl.ds(16 * j, 16)] for j in range(NREG)]

    def compute_chunk(in_ref, out_ref, c):
        @plsc.parallel_loop(0, CHUNK, unroll=2)
        def row_body(r):
            x = [in_ref[r, pl.ds(16 * j, 16)] for j in range(NREG)]
            s_acc = x[0]
            q_acc = x[0] * x[0]
            for j in range(1, NREG):
                s_acc = s_acc + x[j]
                q_acc = q_acc + x[j] * x[j]
            mean_v = _lane_sum(s_acc) * (1.0 / HIDDEN)
            var_v = _lane_sum(q_acc) * (1.0 / HIDDEN) - mean_v * mean_v
            rstd = _rsqrt(var_v + EPS)
            m = plsc.load_gather(
                m_v, [jnp.full((LANES,), c * CHUNK + r, jnp.int32)])
            a = rstd * m
            for j in range(NREG):
                o = (x[j] - mean_v) * (a * g_regs[j]) + b_regs[j] * m
                out_ref[r, pl.ds(16 * j, 16)] = o

    def gather(c, in_ref, sem):
        pltpu.async_copy(table.at[idx_v.at[c]], in_ref, sem)

    def gather_wait(in_ref, sem):
        pltpu.make_async_copy(table.at[idx_v.at[0]], in_ref, sem).wait()

    def writeback(c, out_ref, sem):
        tok0 = pl.multiple_of(base + c * CHUNK, CHUNK)
        pltpu.async_copy(out_ref, out.at[pl.ds(tok0, CHUNK)], sem)

    def wb_wait(out_ref, sem):
        pltpu.make_async_copy(out_ref, out.at[pl.ds(0, CHUNK)], sem).wait()

    gather(0, in0, gs0)

    def pair_body(j, _):
        c0 = 2 * j
        c1 = c0 + 1
        gather_wait(in0, gs0)
        gather(c1, in1, gs1)
        pl.when(j > 0)(lambda: wb_wait(out0, os0))
        compute_chunk(in0, out0, c0)
        writeback(c0, out0, os0)
        gather_wait(in1, gs1)
        pl.when(c0 + 2 < n_chunks)(lambda: gather(c0 + 2, in0, gs0))
        pl.when(j > 0)(lambda: wb_wait(out1, os1))
        compute_chunk(in1, out1, c1)
        writeback(c1, out1, os1)
        return 0

    lax.fori_loop(0, n_chunks // 2, pair_body, 0)
    wb_wait(out0, os0)
    wb_wait(out1, os1)


@jax.jit
def _run(emb_table, ln_gamma, ln_beta, ids_2d, maskf_flat):
    n = maskf_flat.shape[0]
    mesh = plsc.VectorSubcoreMesh(core_axis_name="c", subcore_axis_name="s")
    info = plsc.get_sparse_core_info()
    n_per_w = n // (info.num_cores * info.num_subcores)
    k = pl.kernel(
        _sc_body,
        out_type=jax.ShapeDtypeStruct((n, HIDDEN), jnp.float32),
        mesh=mesh,
        compiler_params=pltpu.CompilerParams(needs_layout_passes=False),
        scratch_types=[
            pltpu.VMEM((n_per_w // CHUNK, CHUNK), jnp.int32),
            pltpu.VMEM((n_per_w,), jnp.float32),
            pltpu.VMEM((CHUNK, HIDDEN), jnp.float32),
            pltpu.VMEM((CHUNK, HIDDEN), jnp.float32),
            pltpu.VMEM((CHUNK, HIDDEN), jnp.float32),
            pltpu.VMEM((CHUNK, HIDDEN), jnp.float32),
            pltpu.VMEM((HIDDEN,), jnp.float32),
            pltpu.VMEM((HIDDEN,), jnp.float32),
            pltpu.SemaphoreType.DMA,
            pltpu.SemaphoreType.DMA,
            pltpu.SemaphoreType.DMA,
            pltpu.SemaphoreType.DMA,
        ],
    )
    return k(emb_table, ids_2d, maskf_flat, ln_gamma, ln_beta)


def kernel(emb_table, ln_gamma, ln_beta, input_ids, attention_mask):
    b, l = input_ids.shape
    ids_2d = input_ids.reshape(b * l // CHUNK, CHUNK)
    maskf_flat = attention_mask.reshape(b * l).astype(jnp.float32)
    out = _run(emb_table, ln_gamma, ln_beta, ids_2d, maskf_flat)
    return out.reshape(b, l, HIDDEN)


# speedup vs baseline: 1.1408x; 1.1408x over previous
import functools

import jax
import jax.numpy as jnp
from jax import lax
from jax.experimental import pallas as pl
from jax.experimental.pallas import tpu as pltpu
from jax.experimental.pallas import tpu_sc as plsc

HIDDEN = 128
EPS = 1e-5
LANES = 16
NREG = HIDDEN // LANES
CHUNK = 128


def _lane_sum(v):
    idx = lax.iota(jnp.int32, LANES)
    dnums = lax.GatherDimensionNumbers(
        offset_dims=(), collapsed_slice_dims=(0,), start_index_map=(0,))
    for sh in (8, 4, 2, 1):
        perm = (idx ^ sh).reshape(LANES, 1)
        v = v + lax.gather(v, perm, dnums, (1,),
                           mode=lax.GatherScatterMode.PROMISE_IN_BOUNDS)
    return v


def _rsqrt(v):
    i = lax.bitcast_convert_type(v, jnp.int32)
    i = jnp.int32(0x5F3759DF) - lax.shift_right_logical(i, 1)
    y = lax.bitcast_convert_type(i, jnp.float32)
    half = v * 0.5
    for _ in range(2):
        y = y * (1.5 - half * y * y)
    return y


def _sc_body(table, ids, maskf, gamma, beta, out,
             idx_v, m_v, in0, in1, out0, out1, g_v, b_v,
             gs0, gs1, os0, os1):
    info = plsc.get_sparse_core_info()
    nc = info.num_cores
    wid = lax.axis_index("s") * nc + lax.axis_index("c")
    n_tok = ids.shape[0] * ids.shape[1]
    n_per_w = n_tok // (nc * info.num_subcores)
    n_chunks = n_per_w // CHUNK
    base = pl.multiple_of(wid * n_per_w, CHUNK)
    base_row = pl.multiple_of(wid * (n_per_w // CHUNK), 8)

    pltpu.sync_copy(gamma, g_v)
    pltpu.sync_copy(beta, b_v)
    pltpu.sync_copy(ids.at[pl.ds(base_row, n_chunks)], idx_v)
    pltpu.sync_copy(maskf.at[pl.ds(base, n_per_w)], m_v)
    g_regs = [g_v[pl.ds(16 * j, 16)] for j in range(NREG)]
    b_regs = [b_v[pl.ds(16 * j, 16)] for j in range(NREG)]

    def compute_chunk(in_ref, out_ref, c):
        @plsc.parallel_loop(0, CHUNK)
        def row_body(r):
            x = [in_ref[r, pl.ds(16 * j, 16)] for j in range(NREG)]
            s_acc = x[0]
            q_acc = x[0] * x[0]
            for j in range(1, NREG):
                s_acc = s_acc + x[j]
                q_acc = q_acc + x[j] * x[j]
            mean_v = _lane_sum(s_acc) * (1.0 / HIDDEN)
            var_v = _lane_sum(q_acc) * (1.0 / HIDDEN) - mean_v * mean_v
            rstd = _rsqrt(var_v + EPS)
            m = plsc.load_gather(
                m_v, [jnp.full((LANES,), c * CHUNK + r, jnp.int32)])
            a = rstd * m
            for j in range(NREG):
                o = (x[j] - mean_v) * (a * g_regs[j]) + b_regs[j] * m
                out_ref[r, pl.ds(16 * j, 16)] = o

    def gather(c, in_ref, sem):
        pltpu.async_copy(table.at[idx_v.at[c]], in_ref, sem)

    def gather_wait(in_ref, sem):
        pltpu.make_async_copy(table.at[idx_v.at[0]], in_ref, sem).wait()

    def writeback(c, out_ref, sem):
        tok0 = pl.multiple_of(base + c * CHUNK, CHUNK)
        pltpu.async_copy(out_ref, out.at[pl.ds(tok0, CHUNK)], sem)

    def wb_wait(out_ref, sem):
        pltpu.make_async_copy(out_ref, out.at[pl.ds(0, CHUNK)], sem).wait()

    gather(0, in0, gs0)

    def pair_body(j, _):
        c0 = 2 * j
        c1 = c0 + 1
        gather_wait(in0, gs0)
        gather(c1, in1, gs1)
        pl.when(j > 0)(lambda: wb_wait(out0, os0))
        compute_chunk(in0, out0, c0)
        writeback(c0, out0, os0)
        gather_wait(in1, gs1)
        pl.when(c0 + 2 < n_chunks)(lambda: gather(c0 + 2, in0, gs0))
        pl.when(j > 0)(lambda: wb_wait(out1, os1))
        compute_chunk(in1, out1, c1)
        writeback(c1, out1, os1)
        return 0

    lax.fori_loop(0, n_chunks // 2, pair_body, 0)
    wb_wait(out0, os0)
    wb_wait(out1, os1)


@jax.jit
def _run(emb_table, ln_gamma, ln_beta, ids_2d, maskf_flat):
    n = maskf_flat.shape[0]
    mesh = plsc.VectorSubcoreMesh(core_axis_name="c", subcore_axis_name="s")
    info = plsc.get_sparse_core_info()
    n_per_w = n // (info.num_cores * info.num_subcores)
    k = pl.kernel(
        _sc_body,
        out_type=jax.ShapeDtypeStruct((n, HIDDEN), jnp.float32),
        mesh=mesh,
        compiler_params=pltpu.CompilerParams(needs_layout_passes=False),
        scratch_types=[
            pltpu.VMEM((n_per_w // CHUNK, CHUNK), jnp.int32),
            pltpu.VMEM((n_per_w,), jnp.float32),
            pltpu.VMEM((CHUNK, HIDDEN), jnp.float32),
            pltpu.VMEM((CHUNK, HIDDEN), jnp.float32),
            pltpu.VMEM((CHUNK, HIDDEN), jnp.float32),
            pltpu.VMEM((CHUNK, HIDDEN), jnp.float32),
            pltpu.VMEM((HIDDEN,), jnp.float32),
            pltpu.VMEM((HIDDEN,), jnp.float32),
            pltpu.SemaphoreType.DMA,
            pltpu.SemaphoreType.DMA,
            pltpu.SemaphoreType.DMA,
            pltpu.SemaphoreType.DMA,
        ],
    )
    return k(emb_table, ids_2d, maskf_flat, ln_gamma, ln_beta)


def kernel(emb_table, ln_gamma, ln_beta, input_ids, attention_mask):
    b, l = input_ids.shape
    ids_2d = input_ids.reshape(b * l // CHUNK, CHUNK)
    maskf_flat = attention_mask.reshape(b * l).astype(jnp.float32)
    out = _run(emb_table, ln_gamma, ln_beta, ids_2d, maskf_flat)
    return out.reshape(b, l, HIDDEN)
